# Initial kernel scaffold; baseline (speedup 1.0000x reference)
import jax
import jax.numpy as jnp
from jax.experimental import pallas as pl


def kernel(s_j, v_j, r_ij, nbrs, ln_gamma, ln_beta, Wq, Wk, Wv, Wdk, bdk, Wdv, bdv, Wd, bd):
    raise NotImplementedError("write your pallas kernel here")


# speedup vs baseline: 14.6258x; 14.6258x over previous
import functools

import jax
import jax.numpy as jnp
from jax import lax
from jax.experimental import pallas as pl
from jax.experimental.pallas import tpu as pltpu
from jax.experimental.pallas import tpu_sc as plsc

EPS = 1e-15
N = 10000
E = 160000
F = 128
H = 4
HF = H * F
NRBF = 20
CUTOFF = 5.0

NC = 2
NS = 16
NW = NC * NS
L = 16


@functools.lru_cache(maxsize=None)
def _sc_mesh():
    return plsc.VectorSubcoreMesh(
        core_axis_name="c", subcore_axis_name="s",
        num_cores=NC, num_subcores=NS)


EPW = E // NW
GCH = 128
G_FULL = EPW // GCH
G_TAIL = EPW - G_FULL * GCH


@functools.lru_cache(maxsize=None)
def _build_sc_gather():
    @functools.partial(
        pl.kernel,
        mesh=_sc_mesh(),
        out_type=(
            jax.ShapeDtypeStruct((E, F), jnp.float32),
            jax.ShapeDtypeStruct((E, F), jnp.float32),
            jax.ShapeDtypeStruct((E, 3 * F), jnp.float32),
        ),
        scratch_types=[
            pltpu.VMEM((GCH,), jnp.int32),
            pltpu.VMEM((GCH,), jnp.int32),
            pltpu.VMEM((GCH, F), jnp.float32),
            pltpu.VMEM((GCH, F), jnp.float32),
            pltpu.VMEM((GCH, 3 * F), jnp.float32),
            pltpu.VMEM((G_TAIL,), jnp.int32),
            pltpu.VMEM((G_TAIL,), jnp.int32),
            pltpu.VMEM((G_TAIL, F), jnp.float32),
            pltpu.VMEM((G_TAIL, F), jnp.float32),
            pltpu.VMEM((G_TAIL, 3 * F), jnp.float32),
            pltpu.SemaphoreType.DMA,
            pltpu.SemaphoreType.DMA,
            pltpu.SemaphoreType.DMA,
        ],
    )
    def _sc_gather(s_hbm, vjf_hbm, ii_hbm, jj_hbm,
                   gi_hbm, gj_hbm, gvj_hbm,
                   ii_v, jj_v, gi_v, gj_v, gvj_v,
                   ii_t, jj_t, gi_t, gj_t, gvj_t,
                   sem1, sem2, sem3):
        wid = lax.axis_index("s") * NC + lax.axis_index("c")
        base0 = wid * EPW

        def do_chunk(base, n, iiv, jjv, giv, gjv, gvjv):
            pltpu.sync_copy(ii_hbm.at[pl.ds(base, n)], iiv)
            pltpu.sync_copy(jj_hbm.at[pl.ds(base, n)], jjv)
            c1 = pltpu.async_copy(s_hbm.at[iiv], giv, sem1)
            c2 = pltpu.async_copy(s_hbm.at[jjv], gjv, sem2)
            c3 = pltpu.async_copy(vjf_hbm.at[jjv], gvjv, sem3)
            c1.wait()
            c2.wait()
            c3.wait()
            pltpu.sync_copy(giv, gi_hbm.at[pl.ds(base, n)])
            pltpu.sync_copy(gjv, gj_hbm.at[pl.ds(base, n)])
            pltpu.sync_copy(gvjv, gvj_hbm.at[pl.ds(base, n)])

        def body(c, carry):
            do_chunk(base0 + c * GCH, GCH, ii_v, jj_v, gi_v, gj_v, gvj_v)
            return carry

        lax.fori_loop(0, G_FULL, body, 0)
        do_chunk(base0 + G_FULL * GCH, G_TAIL,
                 ii_t, jj_t, gi_t, gj_t, gvj_t)

    return _sc_gather


BE = 1000
GRID_E = E // BE


def _sigmoid(x):
    return 1.0 / (1.0 + jnp.exp(-x))


def _edge_body(gi_ref, gj_ref, gvj_ref, r_ref, g_ref, b_ref,
               wq_ref, wk_ref, wv_ref, wdk_ref, bdk_ref, wdv_ref, bdv_ref,
               wd_ref, bd_ref, out_ref):
    g = g_ref[...]
    b = b_ref[...]

    def ln(x):
        m = jnp.mean(x, axis=-1, keepdims=True)
        d = x - m
        v = jnp.mean(d * d, axis=-1, keepdims=True)
        return d / jnp.sqrt(v + 1e-5) * g + b

    li = ln(gi_ref[...])
    lj = ln(gj_ref[...])
    q = jnp.dot(li, wq_ref[...], preferred_element_type=jnp.float32)
    k = jnp.dot(lj, wk_ref[...], preferred_element_type=jnp.float32)
    v = jnp.dot(lj, wv_ref[...], preferred_element_type=jnp.float32)

    r = r_ref[...]
    dist = jnp.sqrt(jnp.sum(r * r + EPS, axis=-1, keepdims=True))
    unit = r / dist
    sigma = CUTOFF / (NRBF - 1)
    mu = lax.broadcasted_iota(jnp.int32, (1, NRBF), 1).astype(jnp.float32) * sigma
    dd = dist - mu
    rbf = jnp.exp(dd * dd * (-1.0 / (2.0 * sigma * sigma)))
    dk = jnp.dot(rbf, wdk_ref[...], preferred_element_type=jnp.float32) + bdk_ref[...]
    dk = dk * _sigmoid(dk)
    dv = jnp.dot(rbf, wdv_ref[...], preferred_element_type=jnp.float32) + bdv_ref[...]
    dv = dv * _sigmoid(dv)

    qkd = q * k * dk
    msg_cols = []
    for h in range(H):
        sl = slice(h * F, (h + 1) * F)
        a = jnp.sum(qkd[:, sl], axis=-1, keepdims=True)
        a = a * _sigmoid(a)
        msg_cols.append(v[:, sl] * dv[:, sl] * a)
    msg = jnp.concatenate(msg_cols, axis=-1)

    inv = jnp.dot(msg, wd_ref[...], preferred_element_type=jnp.float32) + bd_ref[...]
    s0 = inv[:, 0:F]
    s1 = inv[:, F:2 * F]
    s2 = inv[:, 2 * F:3 * F]

    kk3 = lax.broadcasted_iota(jnp.int32, (F, 3 * F), 1)
    ff = lax.broadcasted_iota(jnp.int32, (F, 3 * F), 0)
    R = (kk3 // 3 == ff).astype(jnp.float32)
    kt = lax.broadcasted_iota(jnp.int32, (3, 3 * F), 1)
    cc = lax.broadcasted_iota(jnp.int32, (3, 3 * F), 0)
    T = (kt % 3 == cc).astype(jnp.float32)

    s0_rep = jnp.dot(s0, R, preferred_element_type=jnp.float32)
    s2_rep = jnp.dot(s2, R, preferred_element_type=jnp.float32)
    u_rep = jnp.dot(unit, T, preferred_element_type=jnp.float32)
    dv_out = s2_rep * u_rep + s0_rep * gvj_ref[...]
    out_ref[...] = jnp.concatenate([s1, dv_out], axis=-1)


_edge_call = pl.pallas_call(
    _edge_body,
    grid=(GRID_E,),
    in_specs=[
        pl.BlockSpec((BE, F), lambda e: (e, 0)),
        pl.BlockSpec((BE, F), lambda e: (e, 0)),
        pl.BlockSpec((BE, 3 * F), lambda e: (e, 0)),
        pl.BlockSpec((BE, 3), lambda e: (e, 0)),
        pl.BlockSpec((1, F), lambda e: (0, 0)),
        pl.BlockSpec((1, F), lambda e: (0, 0)),
        pl.BlockSpec((F, HF), lambda e: (0, 0)),
        pl.BlockSpec((F, HF), lambda e: (0, 0)),
        pl.BlockSpec((F, HF), lambda e: (0, 0)),
        pl.BlockSpec((NRBF, HF), lambda e: (0, 0)),
        pl.BlockSpec((1, HF), lambda e: (0, 0)),
        pl.BlockSpec((NRBF, HF), lambda e: (0, 0)),
        pl.BlockSpec((1, HF), lambda e: (0, 0)),
        pl.BlockSpec((HF, 3 * F), lambda e: (0, 0)),
        pl.BlockSpec((1, 3 * F), lambda e: (0, 0)),
    ],
    out_specs=pl.BlockSpec((BE, 4 * F), lambda e: (e, 0)),
    out_shape=jax.ShapeDtypeStruct((E, 4 * F), jnp.float32),
)


EPT = E // NS
SCH = 128
S_FULL = EPT // SCH
S_TAIL = EPT - S_FULL * SCH
NPT = 624
NREM = N - NS * NPT
NREM_OFF = NS * NPT
NZC = NPT // SCH
NZR = NPT - NZC * SCH


@functools.lru_cache(maxsize=None)
def _build_sc_scatter():
    @functools.partial(
        pl.kernel,
        mesh=_sc_mesh(),
        out_type=(
            jax.ShapeDtypeStruct((N, F), jnp.float32),
            jax.ShapeDtypeStruct((N, 3 * F), jnp.float32),
        ),
        scratch_types=[
            pltpu.VMEM((SCH,), jnp.int32),
            pltpu.VMEM((SCH, F), jnp.float32),
            pltpu.VMEM((S_TAIL,), jnp.int32),
            pltpu.VMEM((S_TAIL, F), jnp.float32),
            pltpu.VMEM((SCH, F), jnp.float32),
            pltpu.VMEM_SHARED((N, F), jnp.float32),
        ],
    )
    def _sc_scatter(eo_hbm, ii_hbm, outs_hbm, outv_hbm,
                    idx_v, rows_v, idx_t, rows_t, zb, acc):
        cid = lax.axis_index("c")
        sid = lax.axis_index("s")
        ebase = sid * EPT

        zv = jnp.zeros((L,), jnp.float32)

        def zrow(rr, carry):
            for gg in range(F // L):
                zb[rr, pl.ds(gg * L, L)] = zv
            return carry

        lax.fori_loop(0, SCH, zrow, 0)

        for t in range(2):
            ch = cid * 2 + t
            coloff = ch * F
            rb = sid * NPT
            for zc in range(NZC):
                pltpu.sync_copy(zb, acc.at[pl.ds(rb + zc * SCH, SCH)])
            pltpu.sync_copy(zb.at[pl.ds(0, NZR)],
                            acc.at[pl.ds(rb + NZC * SCH, NZR)])

            @pl.when(sid == NS - 1)
            def _zero_rem():
                pltpu.sync_copy(zb.at[pl.ds(0, NREM)],
                                acc.at[pl.ds(NREM_OFF, NREM)])

            plsc.subcore_barrier()

            def body(c, carry):
                bb = ebase + c * SCH
                pltpu.sync_copy(ii_hbm.at[pl.ds(bb, SCH)], idx_v)
                pltpu.sync_copy(eo_hbm.at[pl.ds(bb, SCH), pl.ds(coloff, F)],
                                rows_v)
                pltpu.sync_copy(rows_v, acc.at[idx_v], add=True)
                return carry

            lax.fori_loop(0, S_FULL, body, 0)
            bb = ebase + S_FULL * SCH
            pltpu.sync_copy(ii_hbm.at[pl.ds(bb, S_TAIL)], idx_t)
            pltpu.sync_copy(eo_hbm.at[pl.ds(bb, S_TAIL), pl.ds(coloff, F)],
                            rows_t)
            pltpu.sync_copy(rows_t, acc.at[idx_t], add=True)
            plsc.subcore_barrier()

            vcol = coloff - F
            if t == 0:
                @pl.when(cid == 0)
                def _ws():
                    pltpu.sync_copy(acc.at[pl.ds(sid * NPT, NPT)],
                                    outs_hbm.at[pl.ds(sid * NPT, NPT)])

                    @pl.when(sid == NS - 1)
                    def _ws_rem():
                        pltpu.sync_copy(acc.at[pl.ds(NREM_OFF, NREM)],
                                        outs_hbm.at[pl.ds(NREM_OFF, NREM)])

                @pl.when(cid != 0)
                def _wv0():
                    pltpu.sync_copy(
                        acc.at[pl.ds(sid * NPT, NPT)],
                        outv_hbm.at[pl.ds(sid * NPT, NPT), pl.ds(vcol, F)])

                    @pl.when(sid == NS - 1)
                    def _wv0_rem():
                        pltpu.sync_copy(
                            acc.at[pl.ds(NREM_OFF, NREM)],
                            outv_hbm.at[pl.ds(NREM_OFF, NREM), pl.ds(vcol, F)])
            else:
                pltpu.sync_copy(
                    acc.at[pl.ds(sid * NPT, NPT)],
                    outv_hbm.at[pl.ds(sid * NPT, NPT), pl.ds(vcol, F)])

                @pl.when(sid == NS - 1)
                def _wv1_rem():
                    pltpu.sync_copy(
                        acc.at[pl.ds(NREM_OFF, NREM)],
                        outv_hbm.at[pl.ds(NREM_OFF, NREM), pl.ds(vcol, F)])

            plsc.subcore_barrier()

    return _sc_scatter


def kernel(s_j, v_j, r_ij, nbrs, ln_gamma, ln_beta, Wq, Wk, Wv,
           Wdk, bdk, Wdv, bdv, Wd, bd):
    nb = nbrs.astype(jnp.int32)
    sel_i = jnp.array([1, 0], jnp.int32)
    sel_j = jnp.array([0, 1], jnp.int32)
    ii = jnp.sum(nb * sel_i[None, :], axis=1, dtype=jnp.int32)
    jj = jnp.sum(nb * sel_j[None, :], axis=1, dtype=jnp.int32)
    vjf = v_j.reshape(N, 3 * F)

    gi, gj, gvj = _build_sc_gather()(s_j, vjf, ii, jj)

    eo = _edge_call(gi, gj, gvj, r_ij,
                    ln_gamma.reshape(1, F), ln_beta.reshape(1, F),
                    Wq, Wk, Wv, Wdk, bdk.reshape(1, HF),
                    Wdv, bdv.reshape(1, HF), Wd, bd.reshape(1, 3 * F))

    delta_s, dvf = _build_sc_scatter()(eo, ii)
    return delta_s, dvf.reshape(N, F, 3)
